# Initial kernel scaffold; baseline (speedup 1.0000x reference)
import jax
import jax.numpy as jnp
from jax.experimental import pallas as pl


def kernel(x, edge_index, batch, emb, Wrel1, brel1, Wroot1, p1, Wrel2, brel2, Wroot2, p2, Wrel3, brel3, Wroot3, p3, Wl1, bl1, g1, be1, Wl2, bl2, g2, be2, Wl3, bl3):
    raise NotImplementedError("write your pallas kernel here")


# speedup vs baseline: 3.1848x; 3.1848x over previous
import functools

import jax
import jax.numpy as jnp
from jax import lax
from jax.experimental import pallas as pl
from jax.experimental.pallas import tpu as pltpu
from jax.experimental.pallas import tpu_sc as plsc

_N = 10000
_E = 320000
_B = 256
_D = 128

_NC = 2
_NS = 16
_NW = _NC * _NS

_CHUNK = 80


def _sc_gather(emb, xv):
    n = xv.shape[0]
    chunks = n // _CHUNK
    per_w = -(-chunks // _NW)

    mesh = plsc.VectorSubcoreMesh(core_axis_name="c", subcore_axis_name="s")

    @functools.partial(
        pl.kernel,
        out_type=jax.ShapeDtypeStruct((n + 8, _D), jnp.float32),
        mesh=mesh,
        scratch_types=[
            pltpu.VMEM((_CHUNK,), jnp.int32),
            pltpu.VMEM((_CHUNK, _D), jnp.float32),
            pltpu.SemaphoreType.DMA,
        ],
    )
    def k(emb_hbm, xv_hbm, out_hbm, idx_v, rows_v, sem):
        c = lax.axis_index("c")
        s = lax.axis_index("s")
        wid = s * _NC + c
        for i in range(per_w):
            cid = wid * per_w + i

            @pl.when(cid < chunks)
            def _():
                base = cid * _CHUNK
                pltpu.sync_copy(xv_hbm.at[pl.ds(base, _CHUNK)], idx_v)
                pltpu.async_copy(emb_hbm.at[idx_v], rows_v, sem).wait()
                pltpu.sync_copy(rows_v, out_hbm.at[pl.ds(base, _CHUNK)])

    return k(emb, xv)


def _sc_edges(h, src, dst):
    e = src.shape[0]
    per_w = e // _NW
    n_ch = per_w // _CHUNK
    rch = _N // _CHUNK
    rch_per_tile = -(-rch // _NS)

    mesh = plsc.VectorSubcoreMesh(core_axis_name="c", subcore_axis_name="s")

    @functools.partial(
        pl.kernel,
        out_type=jax.ShapeDtypeStruct((2 * _N, _D), jnp.float32),
        mesh=mesh,
        scratch_types=[
            pltpu.VMEM((_CHUNK,), jnp.int32),
            pltpu.VMEM((_CHUNK,), jnp.int32),
            pltpu.VMEM((_CHUNK, _D), jnp.float32),
            pltpu.VMEM((_CHUNK, _D), jnp.float32),
            pltpu.VMEM_SHARED((_N, _D), jnp.float32),
            pltpu.SemaphoreType.DMA,
        ],
    )
    def k(h_hbm, src_hbm, dst_hbm, out_hbm, isrc, idst, rows_v, zbuf,
          agg_sh, sem):
        c = lax.axis_index("c")
        s = lax.axis_index("s")
        wid = c * _NS + s

        zero16 = jnp.zeros((16,), jnp.float32)

        def zr(i, carry):
            for j in range(_D // 16):
                zbuf[i, pl.ds(j * 16, 16)] = zero16
            return carry

        lax.fori_loop(0, _CHUNK, zr, 0)
        for i in range(rch_per_tile):
            ch = s + i * _NS

            @pl.when(ch < rch)
            def _():
                pltpu.sync_copy(zbuf, agg_sh.at[pl.ds(ch * _CHUNK, _CHUNK)])
        plsc.subcore_barrier()

        base_e = wid * per_w

        def body(i, carry):
            off = base_e + i * _CHUNK
            pltpu.sync_copy(src_hbm.at[pl.ds(off, _CHUNK)], isrc)
            pltpu.sync_copy(dst_hbm.at[pl.ds(off, _CHUNK)], idst)
            pltpu.async_copy(h_hbm.at[isrc], rows_v, sem).wait()
            pltpu.sync_copy(rows_v, agg_sh.at[idst], add=True)
            return carry

        lax.fori_loop(0, n_ch, body, 0)
        plsc.subcore_barrier()
        for i in range(rch_per_tile):
            ch = s + i * _NS

            @pl.when(ch < rch)
            def _():
                pltpu.sync_copy(
                    agg_sh.at[pl.ds(ch * _CHUNK, _CHUNK)],
                    out_hbm.at[pl.ds(c * _N + ch * _CHUNK, _CHUNK)])

    return k(h, src, dst)


def _tc_layer(aggp, h, Wrel, brel2d, Wroot, pcol, pnrm, off):
    def body(aggp_ref, h_ref, wr_ref, br_ref, wo_ref, p_ref, nrm_ref, off_ref,
             hn_ref, xl_ref):
        a = aggp_ref[...]
        agg = a[:_N] + a[_N:]
        hh = h_ref[...][:_N]
        bf = jnp.bfloat16
        t = (jnp.dot(agg.astype(bf), wr_ref[...].astype(bf),
                     preferred_element_type=jnp.float32)
             + br_ref[...]
             + jnp.dot(hh.astype(bf), wo_ref[...].astype(bf),
                       preferred_element_type=jnp.float32))
        t = jnp.where(t > 0, t, 0.01 * t)
        p = p_ref[...]
        nrm = nrm_ref[...]
        score = jnp.dot(t.astype(bf), p.astype(bf),
                        preferred_element_type=jnp.float32) / (nrm + 1e-16)
        hn = t * jnp.tanh(score)
        hn_ref[:_N, :] = hn
        hn_ref[_N:, :] = jnp.zeros((8, _D), jnp.float32)

        neg = jnp.full((8, _D), -jnp.inf, jnp.float32)
        zero8 = jnp.zeros((8, _D), jnp.float32)

        def seg(g, carry):
            rows_out = []
            for u in range(8):
                b = g * 8 + u
                st = off_ref[b]
                en = off_ref[b + 1]
                nblk = lax.div(en - st + 7, 8)

                def inner(j, mxsm, st=st, en=en):
                    mx, sm = mxsm
                    k0 = st + j * 8
                    rows = hn_ref[pl.ds(k0, 8), :]
                    ridx = k0 + lax.broadcasted_iota(jnp.int32, (8, 1), 0)
                    m = ridx < en
                    mx = jnp.maximum(mx, jnp.where(m, rows, -jnp.inf))
                    sm = sm + jnp.where(m, rows, 0.0)
                    return mx, sm

                mx, sm = lax.fori_loop(0, nblk, inner, (neg, zero8))
                mx1 = jnp.max(mx, axis=0, keepdims=True)
                sm1 = jnp.sum(sm, axis=0, keepdims=True)
                cnt = (en - st).astype(jnp.float32)
                rows_out.append(jnp.concatenate(
                    [jnp.where(cnt > 0, mx1, 0.0),
                     sm1 / jnp.maximum(cnt, 1.0)], axis=1))
            block = jnp.concatenate(rows_out, axis=0)
            xl_ref[pl.ds(pl.multiple_of(g * 8, 8), 8), :] = block
            return carry

        lax.fori_loop(0, _B // 8, seg, 0)

    vm = pl.BlockSpec(memory_space=pltpu.VMEM)
    return pl.pallas_call(
        body,
        in_specs=[vm, vm, vm, vm, vm, vm, vm,
                  pl.BlockSpec(memory_space=pltpu.SMEM)],
        out_specs=(vm, vm),
        out_shape=(jax.ShapeDtypeStruct((_N + 8, _D), jnp.float32),
                   jax.ShapeDtypeStruct((_B, 2 * _D), jnp.float32)),
    )(aggp, h, Wrel, brel2d, Wroot, pcol, pnrm, off)


def _tc_mlp(x1, x2, x3, Wl1, bl1, g1, be1, Wl2, bl2, g2, be2, Wl3, bl3):
    def body(x1r, x2r, x3r, w1r, b1r, g1r, e1r, w2r, b2r, g2r, e2r, w3r, b3r,
             out_ref):
        bf = jnp.bfloat16
        z = x1r[...] + x2r[...] + x3r[...]
        z = jnp.dot(z.astype(bf), w1r[...].astype(bf),
                    preferred_element_type=jnp.float32) + b1r[...]
        z = jnp.where(z > 0, z, 0.01 * z)
        mu = jnp.mean(z, axis=0, keepdims=True)
        d = z - mu
        var = jnp.mean(d * d, axis=0, keepdims=True)
        z = d / jnp.sqrt(var + 1e-5) * g1r[...] + e1r[...]
        z = jnp.dot(z.astype(bf), w2r[...].astype(bf),
                    preferred_element_type=jnp.float32) + b2r[...]
        z = jnp.where(z > 0, z, 0.01 * z)
        mu = jnp.mean(z, axis=0, keepdims=True)
        d = z - mu
        var = jnp.mean(d * d, axis=0, keepdims=True)
        z = d / jnp.sqrt(var + 1e-5) * g2r[...] + e2r[...]
        z = jnp.dot(z.astype(bf), w3r[...].astype(bf),
                    preferred_element_type=jnp.float32) + b3r[...]
        out_ref[...] = z

    vm = pl.BlockSpec(memory_space=pltpu.VMEM)
    return pl.pallas_call(
        body,
        in_specs=[vm] * 13,
        out_specs=vm,
        out_shape=jax.ShapeDtypeStruct((_B, 1), jnp.float32),
    )(x1, x2, x3, Wl1, bl1, g1, be1, Wl2, bl2, g2, be2, Wl3, bl3)


def kernel(x, edge_index, batch, emb, Wrel1, brel1, Wroot1, p1, Wrel2, brel2,
           Wroot2, p2, Wrel3, brel3, Wroot3, p3, Wl1, bl1, g1, be1, Wl2, bl2,
           g2, be2, Wl3, bl3):
    xv = x[:, 0]
    src = edge_index[0]
    dst = edge_index[1]
    perm = jnp.argsort(dst, stable=True)
    src_s = src[perm]
    dst_s = dst[perm]
    off = jnp.searchsorted(
        batch, jnp.arange(_B + 1, dtype=jnp.int32), side="left"
    ).astype(jnp.int32)

    h = _sc_gather(emb, xv)

    xls = []
    for Wrel, brel, Wroot, p in ((Wrel1, brel1, Wroot1, p1),
                                 (Wrel2, brel2, Wroot2, p2),
                                 (Wrel3, brel3, Wroot3, p3)):
        aggp = _sc_edges(h, src_s, dst_s)
        pnrm = jnp.linalg.norm(p).reshape(1, 1)
        h, xl = _tc_layer(aggp, h, Wrel, brel.reshape(1, _D), Wroot,
                          p.reshape(_D, 1), pnrm, off)
        xls.append(xl)

    z = _tc_mlp(xls[0], xls[1], xls[2],
                Wl1, bl1.reshape(1, -1), g1.reshape(1, -1), be1.reshape(1, -1),
                Wl2, bl2.reshape(1, -1), g2.reshape(1, -1), be2.reshape(1, -1),
                Wl3, bl3.reshape(1, -1))
    return z[:, 0]


# speedup vs baseline: 4.3084x; 1.3528x over previous
import functools

import jax
import jax.numpy as jnp
from jax import lax
from jax.experimental import pallas as pl
from jax.experimental.pallas import tpu as pltpu
from jax.experimental.pallas import tpu_sc as plsc

_N = 10000
_E = 320000
_B = 256
_D = 128

_NC = 2
_NS = 16
_NW = _NC * _NS

_CHUNK = 80


def _sc_gather(emb, xv):
    n = xv.shape[0]
    chunks = n // _CHUNK
    per_w = -(-chunks // _NW)

    mesh = plsc.VectorSubcoreMesh(core_axis_name="c", subcore_axis_name="s")

    @functools.partial(
        pl.kernel,
        out_type=jax.ShapeDtypeStruct((n + 8, _D), jnp.float32),
        mesh=mesh,
        scratch_types=[
            pltpu.VMEM((_CHUNK,), jnp.int32),
            pltpu.VMEM((_CHUNK, _D), jnp.float32),
            pltpu.SemaphoreType.DMA,
        ],
    )
    def k(emb_hbm, xv_hbm, out_hbm, idx_v, rows_v, sem):
        c = lax.axis_index("c")
        s = lax.axis_index("s")
        wid = s * _NC + c
        for i in range(per_w):
            cid = wid * per_w + i

            @pl.when(cid < chunks)
            def _():
                base = cid * _CHUNK
                pltpu.sync_copy(xv_hbm.at[pl.ds(base, _CHUNK)], idx_v)
                pltpu.async_copy(emb_hbm.at[idx_v], rows_v, sem).wait()
                pltpu.sync_copy(rows_v, out_hbm.at[pl.ds(base, _CHUNK)])

    return k(emb, xv)


def _sc_edges(h, src, dst):
    e = src.shape[0]
    per_w = e // _NW
    n_ch = per_w // _CHUNK
    rch = _N // _CHUNK
    rch_per_tile = -(-rch // _NS)

    mesh = plsc.VectorSubcoreMesh(core_axis_name="c", subcore_axis_name="s")

    @functools.partial(
        pl.kernel,
        out_type=jax.ShapeDtypeStruct((2 * _N, _D), jnp.float32),
        mesh=mesh,
        scratch_types=[
            pltpu.VMEM((_CHUNK,), jnp.int32),
            pltpu.VMEM((_CHUNK,), jnp.int32),
            pltpu.VMEM((_CHUNK,), jnp.int32),
            pltpu.VMEM((_CHUNK,), jnp.int32),
            pltpu.VMEM((_CHUNK, _D), jnp.float32),
            pltpu.VMEM((_CHUNK, _D), jnp.float32),
            pltpu.VMEM((_CHUNK, _D), jnp.float32),
            pltpu.VMEM_SHARED((_N, _D), jnp.float32),
            pltpu.SemaphoreType.DMA,
            pltpu.SemaphoreType.DMA,
        ],
    )
    def k(h_hbm, src_hbm, dst_hbm, out_hbm, isrc0, isrc1, idst0, idst1,
          rows0, rows1, zbuf, agg_sh, sem0, sem1):
        c = lax.axis_index("c")
        s = lax.axis_index("s")
        wid = c * _NS + s

        zero16 = jnp.zeros((16,), jnp.float32)

        def zr(i, carry):
            for j in range(_D // 16):
                zbuf[i, pl.ds(j * 16, 16)] = zero16
            return carry

        lax.fori_loop(0, _CHUNK, zr, 0)
        for i in range(rch_per_tile):
            ch = s + i * _NS

            @pl.when(ch < rch)
            def _():
                pltpu.sync_copy(zbuf, agg_sh.at[pl.ds(ch * _CHUNK, _CHUNK)])
        plsc.subcore_barrier()

        base_e = wid * per_w
        isrc = (isrc0, isrc1)
        idst = (idst0, idst1)
        rows = (rows0, rows1)
        sems = (sem0, sem1)

        pltpu.sync_copy(src_hbm.at[pl.ds(base_e, _CHUNK)], isrc0)
        pltpu.sync_copy(dst_hbm.at[pl.ds(base_e, _CHUNK)], idst0)
        pltpu.async_copy(h_hbm.at[isrc0], rows0, sem0)

        def body(i, carry):
            cur = lax.rem(i, 2)
            for b in range(2):
                @pl.when(cur == b)
                def _():
                    @pl.when(i + 1 < n_ch)
                    def _():
                        off = base_e + (i + 1) * _CHUNK
                        pltpu.sync_copy(src_hbm.at[pl.ds(off, _CHUNK)],
                                        isrc[1 - b])
                        pltpu.sync_copy(dst_hbm.at[pl.ds(off, _CHUNK)],
                                        idst[1 - b])
                        pltpu.async_copy(h_hbm.at[isrc[1 - b]], rows[1 - b],
                                         sems[1 - b])
                    pltpu.make_async_copy(
                        h_hbm.at[isrc[b]], rows[b], sems[b]).wait()
                    pltpu.sync_copy(rows[b], agg_sh.at[idst[b]], add=True)
            return carry

        lax.fori_loop(0, n_ch, body, 0)
        plsc.subcore_barrier()
        for i in range(rch_per_tile):
            ch = s + i * _NS

            @pl.when(ch < rch)
            def _():
                pltpu.sync_copy(
                    agg_sh.at[pl.ds(ch * _CHUNK, _CHUNK)],
                    out_hbm.at[pl.ds(c * _N + ch * _CHUNK, _CHUNK)])

    return k(h, src, dst)


def _tc_layer(aggp, h, Wrel, brel2d, Wroot, pcol, pnrm, off):
    def body(aggp_ref, h_ref, wr_ref, br_ref, wo_ref, p_ref, nrm_ref, off_ref,
             hn_ref, xl_ref):
        a = aggp_ref[...]
        agg = a[:_N] + a[_N:]
        hh = h_ref[...][:_N]
        bf = jnp.bfloat16
        t = (jnp.dot(agg.astype(bf), wr_ref[...].astype(bf),
                     preferred_element_type=jnp.float32)
             + br_ref[...]
             + jnp.dot(hh.astype(bf), wo_ref[...].astype(bf),
                       preferred_element_type=jnp.float32))
        t = jnp.where(t > 0, t, 0.01 * t)
        p = p_ref[...]
        nrm = nrm_ref[...]
        score = jnp.dot(t.astype(bf), p.astype(bf),
                        preferred_element_type=jnp.float32) / (nrm + 1e-16)
        hn = t * jnp.tanh(score)
        hn_ref[:_N, :] = hn
        hn_ref[_N:, :] = jnp.zeros((8, _D), jnp.float32)

        neg = jnp.full((8, _D), -jnp.inf, jnp.float32)
        zero8 = jnp.zeros((8, _D), jnp.float32)

        def seg(g, carry):
            rows_out = []
            for u in range(8):
                b = g * 8 + u
                st = off_ref[b]
                en = off_ref[b + 1]
                nblk = lax.div(en - st + 7, 8)

                def inner(j, mxsm, st=st, en=en):
                    mx, sm = mxsm
                    k0 = st + j * 8
                    rows = hn_ref[pl.ds(k0, 8), :]
                    ridx = k0 + lax.broadcasted_iota(jnp.int32, (8, 1), 0)
                    m = ridx < en
                    mx = jnp.maximum(mx, jnp.where(m, rows, -jnp.inf))
                    sm = sm + jnp.where(m, rows, 0.0)
                    return mx, sm

                mx, sm = lax.fori_loop(0, nblk, inner, (neg, zero8))
                mx1 = jnp.max(mx, axis=0, keepdims=True)
                sm1 = jnp.sum(sm, axis=0, keepdims=True)
                cnt = (en - st).astype(jnp.float32)
                rows_out.append(jnp.concatenate(
                    [jnp.where(cnt > 0, mx1, 0.0),
                     sm1 / jnp.maximum(cnt, 1.0)], axis=1))
            block = jnp.concatenate(rows_out, axis=0)
            xl_ref[pl.ds(pl.multiple_of(g * 8, 8), 8), :] = block
            return carry

        lax.fori_loop(0, _B // 8, seg, 0)

    vm = pl.BlockSpec(memory_space=pltpu.VMEM)
    return pl.pallas_call(
        body,
        in_specs=[vm, vm, vm, vm, vm, vm, vm,
                  pl.BlockSpec(memory_space=pltpu.SMEM)],
        out_specs=(vm, vm),
        out_shape=(jax.ShapeDtypeStruct((_N + 8, _D), jnp.float32),
                   jax.ShapeDtypeStruct((_B, 2 * _D), jnp.float32)),
    )(aggp, h, Wrel, brel2d, Wroot, pcol, pnrm, off)


def _tc_mlp(x1, x2, x3, Wl1, bl1, g1, be1, Wl2, bl2, g2, be2, Wl3, bl3):
    def body(x1r, x2r, x3r, w1r, b1r, g1r, e1r, w2r, b2r, g2r, e2r, w3r, b3r,
             out_ref):
        bf = jnp.bfloat16
        z = x1r[...] + x2r[...] + x3r[...]
        z = jnp.dot(z.astype(bf), w1r[...].astype(bf),
                    preferred_element_type=jnp.float32) + b1r[...]
        z = jnp.where(z > 0, z, 0.01 * z)
        mu = jnp.mean(z, axis=0, keepdims=True)
        d = z - mu
        var = jnp.mean(d * d, axis=0, keepdims=True)
        z = d / jnp.sqrt(var + 1e-5) * g1r[...] + e1r[...]
        z = jnp.dot(z.astype(bf), w2r[...].astype(bf),
                    preferred_element_type=jnp.float32) + b2r[...]
        z = jnp.where(z > 0, z, 0.01 * z)
        mu = jnp.mean(z, axis=0, keepdims=True)
        d = z - mu
        var = jnp.mean(d * d, axis=0, keepdims=True)
        z = d / jnp.sqrt(var + 1e-5) * g2r[...] + e2r[...]
        z = jnp.dot(z.astype(bf), w3r[...].astype(bf),
                    preferred_element_type=jnp.float32) + b3r[...]
        out_ref[...] = z

    vm = pl.BlockSpec(memory_space=pltpu.VMEM)
    return pl.pallas_call(
        body,
        in_specs=[vm] * 13,
        out_specs=vm,
        out_shape=jax.ShapeDtypeStruct((_B, 1), jnp.float32),
    )(x1, x2, x3, Wl1, bl1, g1, be1, Wl2, bl2, g2, be2, Wl3, bl3)


def kernel(x, edge_index, batch, emb, Wrel1, brel1, Wroot1, p1, Wrel2, brel2,
           Wroot2, p2, Wrel3, brel3, Wroot3, p3, Wl1, bl1, g1, be1, Wl2, bl2,
           g2, be2, Wl3, bl3):
    xv = x[:, 0]
    src = edge_index[0]
    dst = edge_index[1]
    perm = jnp.argsort(dst, stable=True)
    src_s = src[perm]
    dst_s = dst[perm]
    off = jnp.searchsorted(
        batch, jnp.arange(_B + 1, dtype=jnp.int32), side="left"
    ).astype(jnp.int32)

    h = _sc_gather(emb, xv)

    xls = []
    for Wrel, brel, Wroot, p in ((Wrel1, brel1, Wroot1, p1),
                                 (Wrel2, brel2, Wroot2, p2),
                                 (Wrel3, brel3, Wroot3, p3)):
        aggp = _sc_edges(h, src_s, dst_s)
        pnrm = jnp.linalg.norm(p).reshape(1, 1)
        h, xl = _tc_layer(aggp, h, Wrel, brel.reshape(1, _D), Wroot,
                          p.reshape(_D, 1), pnrm, off)
        xls.append(xl)

    z = _tc_mlp(xls[0], xls[1], xls[2],
                Wl1, bl1.reshape(1, -1), g1.reshape(1, -1), be1.reshape(1, -1),
                Wl2, bl2.reshape(1, -1), g2.reshape(1, -1), be2.reshape(1, -1),
                Wl3, bl3.reshape(1, -1))
    return z[:, 0]
